# baseline (speedup 1.0000x reference)
import functools

import jax
import jax.numpy as jnp
from jax import lax
from jax.experimental import pallas as pl
from jax.experimental.pallas import tpu as pltpu
from jax.experimental.pallas import tpu_sc as plsc

_BS = 2000
_V_SPLIT = 80000
_NC = 2
_NS = 16
_L = 16
_CV = 200
_NBUF = 4
_BT = 128


def _tc_body(t_ref, x_ref, o_ref, macc, tacc, *, BS, B):
    i = pl.program_id(0)

    @pl.when(i == 0)
    def _():
        macc[...] = jnp.full((1, B), -jnp.inf, jnp.float32)
        tacc[...] = jnp.full((1, B), -jnp.inf, jnp.float32)

    x = x_ref[...]
    t = t_ref[...]
    idx = jax.lax.broadcasted_iota(jnp.int32, (BS, B), 0) + i * BS
    eq = idx == t
    neg = jnp.float32(-jnp.inf)
    mpart = jnp.max(jnp.where(eq, neg, x), axis=0, keepdims=True)
    tpart = jnp.max(jnp.where(eq, x, neg), axis=0, keepdims=True)
    macc[...] = jnp.maximum(macc[...], mpart)
    tacc[...] = jnp.maximum(tacc[...], tpart)

    @pl.when(i == pl.num_programs(0) - 1)
    def _():
        o_ref[0:1, :] = macc[...]
        o_ref[1:2, :] = tacc[...]


def _sc_body(x_hbm, t_hbm, m_hbm, tv_hbm, tgt_v, buf, res_v, sems,
             *, B, V, V0):
    VS = V - V0
    nq = 4
    span = VS // nq
    nchunk = span // _CV

    c_id = lax.axis_index("c")
    s_id = lax.axis_index("s")
    wid = s_id * _NC + c_id
    bt = wid // nq
    vq = lax.rem(wid, nq)
    b0 = bt * _BT
    v0 = V0 + vq * span

    tcp = pltpu.make_async_copy(
        t_hbm.at[pl.ds(b0, _BT)], tgt_v, sems.at[_NBUF])
    tcp.start()
    tcp.wait()

    neg = jnp.full((_L,), -jnp.inf, jnp.float32)
    tg = [tgt_v[pl.ds(bg * _L, _L)] for bg in range(_BT // _L)]

    def copy(c, slot):
        return pltpu.make_async_copy(
            x_hbm.at[pl.ds(v0 + c * _CV, _CV), pl.ds(b0, _BT)],
            buf.at[slot], sems.at[slot])

    for k in range(_NBUF - 1):
        copy(k, k).start()

    nbg = _BT // _L

    def chunk_step(c, slot, carry):
        copy(c, slot).wait()
        nxt = c + _NBUF - 1

        @pl.when(nxt < nchunk)
        def _():
            copy(nxt, (slot + _NBUF - 1) % _NBUF).start()

        chunk = buf.at[slot]

        def v_body(v, a):
            a = list(a)
            vid = jnp.full((_L,), v0 + c * _CV + v, jnp.int32)
            for bg in range(nbg):
                x16 = chunk[v, pl.ds(bg * _L, _L)]
                veq = tg[bg] == vid
                a[bg] = jnp.maximum(a[bg], jnp.where(veq, neg, x16))
                a[nbg + bg] = jnp.where(veq, x16, a[nbg + bg])
            return tuple(a)

        return lax.fori_loop(0, _CV, v_body, carry)

    def ring_body(q, carry):
        for b in range(_NBUF):
            carry = chunk_step(q * _NBUF + b, b, carry)
        return carry

    init = tuple([neg] * (2 * nbg))
    nfull = (nchunk // _NBUF) * _NBUF
    accs = lax.fori_loop(0, nchunk // _NBUF, ring_body, init)
    for c in range(nfull, nchunk):
        accs = chunk_step(c, c % _NBUF, accs)

    for bg in range(nbg):
        res_v[pl.ds(bg * _L, _L)] = accs[bg]
        res_v[pl.ds(_BT + bg * _L, _L)] = accs[nbg + bg]

    mcp = pltpu.make_async_copy(
        res_v.at[pl.ds(0, _BT)], m_hbm.at[vq, pl.ds(b0, _BT)],
        sems.at[_NBUF])
    mcp.start()
    mcp.wait()
    tvcp = pltpu.make_async_copy(
        res_v.at[pl.ds(_BT, _BT)], tv_hbm.at[vq, pl.ds(b0, _BT)],
        sems.at[_NBUF])
    tvcp.start()
    tvcp.wait()


def kernel(logits, target):
    B, V = logits.shape
    xt = logits.T
    t32 = target.astype(jnp.int32)
    t2 = t32.reshape(1, B)

    tc_out = pl.pallas_call(
        functools.partial(_tc_body, BS=_BS, B=B),
        grid=(_V_SPLIT // _BS,),
        in_specs=[
            pl.BlockSpec((1, B), lambda i: (0, 0)),
            pl.BlockSpec((_BS, B), lambda i: (i, 0)),
        ],
        out_specs=pl.BlockSpec((2, B), lambda i: (0, 0)),
        out_shape=jax.ShapeDtypeStruct((2, B), jnp.float32),
        scratch_shapes=[
            pltpu.VMEM((1, B), jnp.float32),
            pltpu.VMEM((1, B), jnp.float32),
        ],
    )(t2, xt)

    mesh = plsc.VectorSubcoreMesh(core_axis_name="c", subcore_axis_name="s")
    sc_run = pl.kernel(
        functools.partial(_sc_body, B=B, V=V, V0=_V_SPLIT),
        out_type=(
            jax.ShapeDtypeStruct((4, B), jnp.float32),
            jax.ShapeDtypeStruct((4, B), jnp.float32),
        ),
        mesh=mesh,
        compiler_params=pltpu.CompilerParams(needs_layout_passes=False),
        scratch_types=[
            pltpu.VMEM((_BT,), jnp.int32),
            pltpu.VMEM((_NBUF, _CV, _BT), jnp.float32),
            pltpu.VMEM((2 * _BT,), jnp.float32),
            pltpu.SemaphoreType.DMA((_NBUF + 1,)),
        ],
    )
    sc_m, sc_tv = sc_run(xt, t32)

    m = jnp.maximum(tc_out[0], jnp.max(sc_m, axis=0))
    tv = jnp.maximum(tc_out[1], jnp.max(sc_tv, axis=0))
    return m - tv
